# Initial kernel scaffold; baseline (speedup 1.0000x reference)
import jax
import jax.numpy as jnp
from jax import lax
from jax.experimental import pallas as pl
from jax.experimental.pallas import tpu as pltpu
from jax.experimental.pallas import tpu_sc as plsc

T, N, E, D, H, R = 4, 10000, 320000, 128, 128, 16
NC, NS = 2, 16
NW = NC * NS
EPW = E // NW
BB = 80
NBLK = EPW // BB
EPS = E // NC
ZB = N // NS
LG = H // 16


def _gather_add_body(p_tab, q_tab, ridx, cidx, g_out,
                     idx_r, idx_c, buf_p, buf_q, sem_p, sem_q):
    cid = lax.axis_index("c")
    sid = lax.axis_index("s")
    wid = sid * NC + cid
    for t in range(T):
        base_t = t * E + wid * EPW
        pltpu.sync_copy(ridx.at[pl.ds(base_t, EPW)], idx_r)
        pltpu.sync_copy(cidx.at[pl.ds(base_t, EPW)], idx_c)

        def blk(b, carry):
            eb = b * BB
            cp_p = pltpu.async_copy(p_tab.at[idx_r.at[pl.ds(eb, BB)]],
                                    buf_p, sem_p)
            cp_q = pltpu.async_copy(q_tab.at[idx_c.at[pl.ds(eb, BB)]],
                                    buf_q, sem_q)
            cp_p.wait()
            cp_q.wait()

            def rowadd(r, c2):
                for j in range(LG):
                    s = pl.ds(j * 16, 16)
                    buf_p[r, s] = buf_p[r, s] + buf_q[r, s]
                return c2

            lax.fori_loop(0, BB, rowadd, 0)
            pltpu.sync_copy(buf_p, g_out.at[pl.ds(base_t + eb, BB)])
            return carry

        lax.fori_loop(0, NBLK, blk, 0)


def _gather_add(p_tab, q_tab, ridx, cidx):
    return pl.kernel(
        _gather_add_body,
        out_type=jax.ShapeDtypeStruct((T * E, H), jnp.float32),
        mesh=plsc.VectorSubcoreMesh(core_axis_name="c", subcore_axis_name="s"),
        scratch_types=[
            pltpu.VMEM((EPW,), jnp.int32),
            pltpu.VMEM((EPW,), jnp.int32),
            pltpu.VMEM((BB, H), jnp.float32),
            pltpu.VMEM((BB, H), jnp.float32),
            pltpu.SemaphoreType.DMA,
            pltpu.SemaphoreType.DMA,
        ],
    )(p_tab, q_tab, ridx, cidx)


def _scatter_body(e2, row2d, parts, idx2d, buf, zbuf, acc):
    cid = lax.axis_index("c")
    sid = lax.axis_index("s")
    pltpu.sync_copy(row2d.at[pl.ds(cid * (EPS // BB) + sid * NBLK, NBLK)],
                    idx2d)

    def zrow(r, c):
        for j in range(LG):
            zbuf[r, pl.ds(j * 16, 16)] = jnp.zeros((16,), jnp.float32)
        return c

    lax.fori_loop(0, ZB, zrow, 0)

    for t in range(T):
        pltpu.sync_copy(zbuf, acc.at[pl.ds(sid * ZB, ZB)])
        plsc.subcore_barrier()

        def blk(b, c):
            base = cid * EPS + sid * EPW + b * BB
            pltpu.sync_copy(e2.at[pl.ds(t * E + base, BB)], buf)
            pltpu.sync_copy(buf, acc.at[idx2d.at[b]], add=True)
            return c

        lax.fori_loop(0, NBLK, blk, 0)
        plsc.subcore_barrier()
        pltpu.sync_copy(acc.at[pl.ds(sid * ZB, ZB)],
                        parts.at[t, cid, pl.ds(sid * ZB, ZB)])
        plsc.subcore_barrier()


def _scatter(e2, row2d):
    return pl.kernel(
        _scatter_body,
        out_type=jax.ShapeDtypeStruct((T, NC, N, H), jnp.float32),
        mesh=plsc.VectorSubcoreMesh(core_axis_name="c", subcore_axis_name="s"),
        scratch_types=[
            pltpu.VMEM((NBLK, BB), jnp.int32),
            pltpu.VMEM((BB, H), jnp.float32),
            pltpu.VMEM((ZB, H), jnp.float32),
            pltpu.VMEM_SHARED((N, H), jnp.float32),
        ],
    )(e2, row2d)


BA = 2000


def _tables_tc(h_ref, wa_ref, wb_ref, p_ref, q_ref):
    hb = h_ref[...]
    p_ref[...] = jnp.dot(hb, wa_ref[...], preferred_element_type=jnp.float32)
    q_ref[...] = jnp.dot(hb, wb_ref[...], preferred_element_type=jnp.float32)


def _tables(h2d, wa, wb):
    return pl.pallas_call(
        _tables_tc,
        grid=(T * N // BA,),
        in_specs=[
            pl.BlockSpec((BA, D), lambda i: (i, 0)),
            pl.BlockSpec((D, H), lambda i: (0, 0)),
            pl.BlockSpec((D, H), lambda i: (0, 0)),
        ],
        out_specs=[pl.BlockSpec((BA, H), lambda i: (i, 0))] * 2,
        out_shape=[jax.ShapeDtypeStruct((T * N, H), jnp.float32)] * 2,
    )(h2d, wa, wb)


BC = 4000


def _edge_tc(g_ref, rad_ref, wc_ref, b1_ref, w2_ref, b2_ref, e2_ref):
    e1 = (g_ref[...]
          + jnp.dot(rad_ref[...], wc_ref[...],
                    preferred_element_type=jnp.float32)
          + b1_ref[...])
    e1 = jnp.maximum(e1, 0.0)
    e2 = jnp.dot(e1, w2_ref[...], preferred_element_type=jnp.float32)
    e2_ref[...] = jnp.maximum(e2 + b2_ref[...], 0.0)


def _edge_mlp(g, rad2d, wc, b1, w2, b2):
    return pl.pallas_call(
        _edge_tc,
        grid=(T * E // BC,),
        in_specs=[
            pl.BlockSpec((BC, H), lambda i: (i, 0)),
            pl.BlockSpec((BC, R), lambda i: (i, 0)),
            pl.BlockSpec((R, H), lambda i: (0, 0)),
            pl.BlockSpec((1, H), lambda i: (0, 0)),
            pl.BlockSpec((H, H), lambda i: (0, 0)),
            pl.BlockSpec((1, H), lambda i: (0, 0)),
        ],
        out_specs=pl.BlockSpec((BC, H), lambda i: (i, 0)),
        out_shape=jax.ShapeDtypeStruct((T * E, H), jnp.float32),
    )(g, rad2d, wc, b1, w2, b2)


BN = 2000


def _node_tc(oth_ref, h_ref, parts_ref, wn1_ref, bn1_ref, wn2_ref, bn2_ref,
             hout_ref, a_ref):
    oth = oth_ref[0]
    hb = h_ref[0]
    agg = parts_ref[0, 0] + parts_ref[0, 1]
    a = jnp.concatenate([oth, hb, agg], axis=1)
    a_ref[0] = a
    z = jnp.maximum(
        jnp.dot(a, wn1_ref[...], preferred_element_type=jnp.float32)
        + bn1_ref[...], 0.0)
    o = jnp.dot(z, wn2_ref[...], preferred_element_type=jnp.float32) \
        + bn2_ref[...]
    hout_ref[0] = hb + o


def _node_mlp(others, h, parts, wn1, bn1, wn2, bn2):
    return pl.pallas_call(
        _node_tc,
        grid=(T, N // BN),
        in_specs=[
            pl.BlockSpec((1, BN, H), lambda t, j: (t, j, 0)),
            pl.BlockSpec((1, BN, D), lambda t, j: (t, j, 0)),
            pl.BlockSpec((1, NC, BN, H), lambda t, j: (t, 0, j, 0)),
            pl.BlockSpec((H + D + H, H), lambda t, j: (0, 0)),
            pl.BlockSpec((1, H), lambda t, j: (0, 0)),
            pl.BlockSpec((H, D), lambda t, j: (0, 0)),
            pl.BlockSpec((1, D), lambda t, j: (0, 0)),
        ],
        out_specs=[
            pl.BlockSpec((1, BN, D), lambda t, j: (t, j, 0)),
            pl.BlockSpec((1, BN, H + D + H), lambda t, j: (t, j, 0)),
        ],
        out_shape=[
            jax.ShapeDtypeStruct((T, N, D), jnp.float32),
            jax.ShapeDtypeStruct((T, N, H + D + H), jnp.float32),
        ],
    )(others, h, parts, wn1, bn1, wn2, bn2)


def kernel(h, edge_index, radial, others,
           We1, be1, We2, be2, Wn1, bn1, Wn2, bn2):
    row = edge_index[0]
    col = edge_index[1]
    offs = (jnp.arange(T, dtype=jnp.int32) * N)[:, None]
    ridx = (row[None, :] + offs).reshape(T * E)
    cidx = (col[None, :] + offs).reshape(T * E)
    row2d = row.reshape(E // BB, BB)

    h2d = h.reshape(T * N, D)
    rad2d = radial.reshape(T * E, R)

    p_tab, q_tab = _tables(h2d, We1[:D], We1[D:2 * D])
    g = _gather_add(p_tab, q_tab, ridx, cidx)
    e2 = _edge_mlp(g, rad2d, We1[2 * D:], be1.reshape(1, H), We2,
                   be2.reshape(1, H))
    parts = _scatter(e2, row2d)
    h_out, a_out = _node_mlp(others, h, parts, Wn1, bn1.reshape(1, H), Wn2,
                             bn2.reshape(1, D))
    return h_out, a_out


# speedup vs baseline: 12.9970x; 12.9970x over previous
import jax
import jax.numpy as jnp
from jax import lax
from jax.experimental import pallas as pl
from jax.experimental.pallas import tpu as pltpu
from jax.experimental.pallas import tpu_sc as plsc

T, N, E, D, H, R = 4, 10000, 320000, 128, 128, 16
NC, NS = 2, 16
NW = NC * NS
EPW = E // NW
BB = 80
NBLK = EPW // BB
EPS = E // NC
ZCH = 40
ZTW = 10
ZPT = N // ZTW
LG = H // 16


def _gather_add_body(p_tab, q_tab, ridx, cidx, g_out,
                     idx_r, idx_c, buf_p, buf_q, sem_p, sem_q):
    cid = lax.axis_index("c")
    sid = lax.axis_index("s")
    wid = sid * NC + cid
    for t in range(T):
        base_t = t * E + wid * EPW
        pltpu.sync_copy(ridx.at[pl.ds(base_t, EPW)], idx_r)
        pltpu.sync_copy(cidx.at[pl.ds(base_t, EPW)], idx_c)

        def blk(b, carry):
            eb = b * BB
            cp_p = pltpu.async_copy(p_tab.at[idx_r.at[pl.ds(eb, BB)]],
                                    buf_p, sem_p)
            cp_q = pltpu.async_copy(q_tab.at[idx_c.at[pl.ds(eb, BB)]],
                                    buf_q, sem_q)
            cp_p.wait()
            cp_q.wait()

            def rowadd(r, c2):
                for j in range(LG):
                    s = pl.ds(j * 16, 16)
                    buf_p[r, s] = buf_p[r, s] + buf_q[r, s]
                return c2

            lax.fori_loop(0, BB, rowadd, 0)
            pltpu.sync_copy(buf_p, g_out.at[pl.ds(base_t + eb, BB)])
            return carry

        lax.fori_loop(0, NBLK, blk, 0)


def _gather_add(p_tab, q_tab, ridx, cidx):
    return pl.kernel(
        _gather_add_body,
        out_type=jax.ShapeDtypeStruct((T * E, H), jnp.float32),
        mesh=plsc.VectorSubcoreMesh(core_axis_name="c", subcore_axis_name="s"),
        scratch_types=[
            pltpu.VMEM((EPW,), jnp.int32),
            pltpu.VMEM((EPW,), jnp.int32),
            pltpu.VMEM((BB, H), jnp.float32),
            pltpu.VMEM((BB, H), jnp.float32),
            pltpu.SemaphoreType.DMA,
            pltpu.SemaphoreType.DMA,
        ],
    )(p_tab, q_tab, ridx, cidx)


def _scatter_body(e2, row3d, parts, idx2d, buf, zbuf, acc):
    cid = lax.axis_index("c")
    sid = lax.axis_index("s")
    w = cid * NS + sid
    pltpu.sync_copy(row3d.at[w], idx2d)

    def zrow(r, c):
        for j in range(LG):
            zbuf[r, pl.ds(j * 16, 16)] = jnp.zeros((16,), jnp.float32)
        return c

    lax.fori_loop(0, ZCH, zrow, 0)

    for t in range(T):
        @pl.when(sid < ZTW)
        def _zero():
            def zc(k, c):
                pltpu.sync_copy(zbuf,
                                acc.at[pl.ds(sid * ZPT + k * ZCH, ZCH)])
                return c
            lax.fori_loop(0, ZPT // ZCH, zc, 0)

        plsc.subcore_barrier()

        def blk(b, c):
            base = cid * EPS + sid * EPW + b * BB
            pltpu.sync_copy(e2.at[pl.ds(t * E + base, BB)], buf)
            pltpu.sync_copy(buf, acc.at[idx2d.at[b]], add=True)
            return c

        lax.fori_loop(0, NBLK, blk, 0)
        plsc.subcore_barrier()

        @pl.when(sid < ZTW)
        def _readback():
            def rb(k, c):
                off = sid * ZPT + k * ZCH
                pltpu.sync_copy(acc.at[pl.ds(off, ZCH)],
                                parts.at[t, cid, pl.ds(off, ZCH)])
                return c
            lax.fori_loop(0, ZPT // ZCH, rb, 0)

        plsc.subcore_barrier()


def _scatter(e2, row3d):
    return pl.kernel(
        _scatter_body,
        out_type=jax.ShapeDtypeStruct((T, NC, N, H), jnp.float32),
        mesh=plsc.VectorSubcoreMesh(core_axis_name="c", subcore_axis_name="s"),
        scratch_types=[
            pltpu.VMEM((NBLK, BB), jnp.int32),
            pltpu.VMEM((BB, H), jnp.float32),
            pltpu.VMEM((ZCH, H), jnp.float32),
            pltpu.VMEM_SHARED((N, H), jnp.float32),
        ],
    )(e2, row3d)


BA = 2000


def _tables_tc(h_ref, wa_ref, wb_ref, p_ref, q_ref):
    hb = h_ref[...]
    p_ref[...] = jnp.dot(hb, wa_ref[...], preferred_element_type=jnp.float32)
    q_ref[...] = jnp.dot(hb, wb_ref[...], preferred_element_type=jnp.float32)


def _tables(h2d, wa, wb):
    return pl.pallas_call(
        _tables_tc,
        grid=(T * N // BA,),
        in_specs=[
            pl.BlockSpec((BA, D), lambda i: (i, 0)),
            pl.BlockSpec((D, H), lambda i: (0, 0)),
            pl.BlockSpec((D, H), lambda i: (0, 0)),
        ],
        out_specs=[pl.BlockSpec((BA, H), lambda i: (i, 0))] * 2,
        out_shape=[jax.ShapeDtypeStruct((T * N, H), jnp.float32)] * 2,
    )(h2d, wa, wb)


BC = 4000


def _edge_tc(g_ref, rad_ref, wc_ref, b1_ref, w2_ref, b2_ref, e2_ref):
    e1 = (g_ref[...]
          + jnp.dot(rad_ref[...], wc_ref[...],
                    preferred_element_type=jnp.float32)
          + b1_ref[...])
    e1 = jnp.maximum(e1, 0.0)
    e2 = jnp.dot(e1, w2_ref[...], preferred_element_type=jnp.float32)
    e2_ref[...] = jnp.maximum(e2 + b2_ref[...], 0.0)


def _edge_mlp(g, rad2d, wc, b1, w2, b2):
    return pl.pallas_call(
        _edge_tc,
        grid=(T * E // BC,),
        in_specs=[
            pl.BlockSpec((BC, H), lambda i: (i, 0)),
            pl.BlockSpec((BC, R), lambda i: (i, 0)),
            pl.BlockSpec((R, H), lambda i: (0, 0)),
            pl.BlockSpec((1, H), lambda i: (0, 0)),
            pl.BlockSpec((H, H), lambda i: (0, 0)),
            pl.BlockSpec((1, H), lambda i: (0, 0)),
        ],
        out_specs=pl.BlockSpec((BC, H), lambda i: (i, 0)),
        out_shape=jax.ShapeDtypeStruct((T * E, H), jnp.float32),
    )(g, rad2d, wc, b1, w2, b2)


BN = 2000


def _node_tc(oth_ref, h_ref, parts_ref, wn1_ref, bn1_ref, wn2_ref, bn2_ref,
             hout_ref, a_ref):
    oth = oth_ref[0]
    hb = h_ref[0]
    agg = parts_ref[0, 0] + parts_ref[0, 1]
    a = jnp.concatenate([oth, hb, agg], axis=1)
    a_ref[0] = a
    z = jnp.maximum(
        jnp.dot(a, wn1_ref[...], preferred_element_type=jnp.float32)
        + bn1_ref[...], 0.0)
    o = jnp.dot(z, wn2_ref[...], preferred_element_type=jnp.float32) \
        + bn2_ref[...]
    hout_ref[0] = hb + o


def _node_mlp(others, h, parts, wn1, bn1, wn2, bn2):
    return pl.pallas_call(
        _node_tc,
        grid=(T, N // BN),
        in_specs=[
            pl.BlockSpec((1, BN, H), lambda t, j: (t, j, 0)),
            pl.BlockSpec((1, BN, D), lambda t, j: (t, j, 0)),
            pl.BlockSpec((1, NC, BN, H), lambda t, j: (t, 0, j, 0)),
            pl.BlockSpec((H + D + H, H), lambda t, j: (0, 0)),
            pl.BlockSpec((1, H), lambda t, j: (0, 0)),
            pl.BlockSpec((H, D), lambda t, j: (0, 0)),
            pl.BlockSpec((1, D), lambda t, j: (0, 0)),
        ],
        out_specs=[
            pl.BlockSpec((1, BN, D), lambda t, j: (t, j, 0)),
            pl.BlockSpec((1, BN, H + D + H), lambda t, j: (t, j, 0)),
        ],
        out_shape=[
            jax.ShapeDtypeStruct((T, N, D), jnp.float32),
            jax.ShapeDtypeStruct((T, N, H + D + H), jnp.float32),
        ],
    )(others, h, parts, wn1, bn1, wn2, bn2)


def kernel(h, edge_index, radial, others,
           We1, be1, We2, be2, Wn1, bn1, Wn2, bn2):
    row = edge_index[0]
    col = edge_index[1]
    offs = (jnp.arange(T, dtype=jnp.int32) * N)[:, None]
    ridx = (row[None, :] + offs).reshape(T * E)
    cidx = (col[None, :] + offs).reshape(T * E)
    row3d = row.reshape(NW, NBLK, BB)

    h2d = h.reshape(T * N, D)
    rad2d = radial.reshape(T * E, R)

    p_tab, q_tab = _tables(h2d, We1[:D], We1[D:2 * D])
    g = _gather_add(p_tab, q_tab, ridx, cidx)
    e2 = _edge_mlp(g, rad2d, We1[2 * D:], be1.reshape(1, H), We2,
                   be2.reshape(1, H))
    parts = _scatter(e2, row3d)
    h_out, a_out = _node_mlp(others, h, parts, Wn1, bn1.reshape(1, H), Wn2,
                             bn2.reshape(1, D))
    return h_out, a_out


# speedup vs baseline: 18.4932x; 1.4229x over previous
import jax
import jax.numpy as jnp
from jax import lax
from jax.experimental import pallas as pl
from jax.experimental.pallas import tpu as pltpu
from jax.experimental.pallas import tpu_sc as plsc

T, N, E, D, H, R = 4, 10000, 320000, 128, 128, 16
NC, NS = 2, 16
NW = NC * NS
EPW = E // NW
BB = 80
NBLK = EPW // BB
EPS = E // NC
ZCH = 40
ZTW = 10
ZPT = N // ZTW
LG = H // 16


def _gather_add_body(p_tab, q_tab, ridx, cidx, g_out,
                     idx_r, idx_c, bp0, bq0, bp1, bq1, sem0, sem1):
    cid = lax.axis_index("c")
    sid = lax.axis_index("s")
    wid = sid * NC + cid

    def fire(bp, bq, sem, eb):
        pltpu.async_copy(p_tab.at[idx_r.at[pl.ds(eb, BB)]], bp, sem)
        pltpu.async_copy(q_tab.at[idx_c.at[pl.ds(eb, BB)]], bq, sem)

    def drain(bp, bq, sem):
        pltpu.make_async_copy(p_tab.at[idx_r.at[pl.ds(0, BB)]],
                              bp, sem).wait()
        pltpu.make_async_copy(q_tab.at[idx_c.at[pl.ds(0, BB)]],
                              bq, sem).wait()

    def add_store(bp, bq, base_out):
        def rowadd(r, c2):
            for j in range(LG):
                s = pl.ds(j * 16, 16)
                bp[r, s] = bp[r, s] + bq[r, s]
            return c2
        lax.fori_loop(0, BB, rowadd, 0)
        pltpu.sync_copy(bp, g_out.at[pl.ds(base_out, BB)])

    for t in range(T):
        base_t = t * E + wid * EPW
        pltpu.sync_copy(ridx.at[pl.ds(base_t, EPW)], idx_r)
        pltpu.sync_copy(cidx.at[pl.ds(base_t, EPW)], idx_c)
        fire(bp0, bq0, sem0, 0)

        def pair(k, c):
            eb0 = 2 * k * BB
            fire(bp1, bq1, sem1, eb0 + BB)
            drain(bp0, bq0, sem0)
            add_store(bp0, bq0, base_t + eb0)
            fire(bp0, bq0, sem0, eb0 + 2 * BB)
            drain(bp1, bq1, sem1)
            add_store(bp1, bq1, base_t + eb0 + BB)
            return c

        lax.fori_loop(0, (NBLK - 1) // 2, pair, 0)
        drain(bp0, bq0, sem0)
        add_store(bp0, bq0, base_t + (NBLK - 1) * BB)


def _gather_add(p_tab, q_tab, ridx, cidx):
    return pl.kernel(
        _gather_add_body,
        out_type=jax.ShapeDtypeStruct((T * E, H), jnp.float32),
        mesh=plsc.VectorSubcoreMesh(core_axis_name="c", subcore_axis_name="s"),
        scratch_types=[
            pltpu.VMEM((EPW,), jnp.int32),
            pltpu.VMEM((EPW,), jnp.int32),
            pltpu.VMEM((BB, H), jnp.float32),
            pltpu.VMEM((BB, H), jnp.float32),
            pltpu.VMEM((BB, H), jnp.float32),
            pltpu.VMEM((BB, H), jnp.float32),
            pltpu.SemaphoreType.DMA,
            pltpu.SemaphoreType.DMA,
        ],
    )(p_tab, q_tab, ridx, cidx)


def _scatter_body(e2, row3d, parts, idx2d, buf, buf1, zbuf, acc, sem0, sem1):
    cid = lax.axis_index("c")
    sid = lax.axis_index("s")
    w = cid * NS + sid
    pltpu.sync_copy(row3d.at[w], idx2d)

    def zrow(r, c):
        for j in range(LG):
            zbuf[r, pl.ds(j * 16, 16)] = jnp.zeros((16,), jnp.float32)
        return c

    lax.fori_loop(0, ZCH, zrow, 0)

    tbase = cid * EPS + sid * EPW

    def fire(b, bf, sem, t):
        pltpu.async_copy(e2.at[pl.ds(t * E + tbase + b * BB, BB)], bf, sem)

    def drain(bf, sem, t):
        pltpu.make_async_copy(e2.at[pl.ds(t * E + tbase, BB)], bf, sem).wait()

    for t in range(T):
        @pl.when(sid < ZTW)
        def _zero():
            def zc(k, c):
                pltpu.sync_copy(zbuf,
                                acc.at[pl.ds(sid * ZPT + k * ZCH, ZCH)])
                return c
            lax.fori_loop(0, ZPT // ZCH, zc, 0)

        plsc.subcore_barrier()
        fire(0, buf, sem0, t)

        def pair(k, c):
            b0 = 2 * k
            fire(b0 + 1, buf1, sem1, t)
            drain(buf, sem0, t)
            pltpu.sync_copy(buf, acc.at[idx2d.at[b0]], add=True)
            fire(b0 + 2, buf, sem0, t)
            drain(buf1, sem1, t)
            pltpu.sync_copy(buf1, acc.at[idx2d.at[b0 + 1]], add=True)
            return c

        lax.fori_loop(0, (NBLK - 1) // 2, pair, 0)
        drain(buf, sem0, t)
        pltpu.sync_copy(buf, acc.at[idx2d.at[NBLK - 1]], add=True)
        plsc.subcore_barrier()

        @pl.when(sid < ZTW)
        def _readback():
            def rb(k, c):
                off = sid * ZPT + k * ZCH
                pltpu.sync_copy(acc.at[pl.ds(off, ZCH)],
                                parts.at[t, cid, pl.ds(off, ZCH)])
                return c
            lax.fori_loop(0, ZPT // ZCH, rb, 0)

        plsc.subcore_barrier()


def _scatter(e2, row3d):
    return pl.kernel(
        _scatter_body,
        out_type=jax.ShapeDtypeStruct((T, NC, N, H), jnp.float32),
        mesh=plsc.VectorSubcoreMesh(core_axis_name="c", subcore_axis_name="s"),
        scratch_types=[
            pltpu.VMEM((NBLK, BB), jnp.int32),
            pltpu.VMEM((BB, H), jnp.float32),
            pltpu.VMEM((BB, H), jnp.float32),
            pltpu.VMEM((ZCH, H), jnp.float32),
            pltpu.VMEM_SHARED((N, H), jnp.float32),
            pltpu.SemaphoreType.DMA,
            pltpu.SemaphoreType.DMA,
        ],
    )(e2, row3d)


BA = 2000


def _tables_tc(h_ref, wa_ref, wb_ref, p_ref, q_ref):
    hb = h_ref[...]
    p_ref[...] = jnp.dot(hb, wa_ref[...], preferred_element_type=jnp.float32)
    q_ref[...] = jnp.dot(hb, wb_ref[...], preferred_element_type=jnp.float32)


def _tables(h2d, wa, wb):
    return pl.pallas_call(
        _tables_tc,
        grid=(T * N // BA,),
        in_specs=[
            pl.BlockSpec((BA, D), lambda i: (i, 0)),
            pl.BlockSpec((D, H), lambda i: (0, 0)),
            pl.BlockSpec((D, H), lambda i: (0, 0)),
        ],
        out_specs=[pl.BlockSpec((BA, H), lambda i: (i, 0))] * 2,
        out_shape=[jax.ShapeDtypeStruct((T * N, H), jnp.float32)] * 2,
    )(h2d, wa, wb)


BC = 4000


def _edge_tc(g_ref, rad_ref, wc_ref, b1_ref, w2_ref, b2_ref, e2_ref):
    e1 = (g_ref[...]
          + jnp.dot(rad_ref[...], wc_ref[...],
                    preferred_element_type=jnp.float32)
          + b1_ref[...])
    e1 = jnp.maximum(e1, 0.0)
    e2 = jnp.dot(e1, w2_ref[...], preferred_element_type=jnp.float32)
    e2_ref[...] = jnp.maximum(e2 + b2_ref[...], 0.0)


def _edge_mlp(g, rad2d, wc, b1, w2, b2):
    return pl.pallas_call(
        _edge_tc,
        grid=(T * E // BC,),
        in_specs=[
            pl.BlockSpec((BC, H), lambda i: (i, 0)),
            pl.BlockSpec((BC, R), lambda i: (i, 0)),
            pl.BlockSpec((R, H), lambda i: (0, 0)),
            pl.BlockSpec((1, H), lambda i: (0, 0)),
            pl.BlockSpec((H, H), lambda i: (0, 0)),
            pl.BlockSpec((1, H), lambda i: (0, 0)),
        ],
        out_specs=pl.BlockSpec((BC, H), lambda i: (i, 0)),
        out_shape=jax.ShapeDtypeStruct((T * E, H), jnp.float32),
    )(g, rad2d, wc, b1, w2, b2)


BN = 2000


def _node_tc(oth_ref, h_ref, parts_ref, wn1_ref, bn1_ref, wn2_ref, bn2_ref,
             hout_ref, a_ref):
    oth = oth_ref[0]
    hb = h_ref[0]
    agg = parts_ref[0, 0] + parts_ref[0, 1]
    a = jnp.concatenate([oth, hb, agg], axis=1)
    a_ref[0] = a
    z = jnp.maximum(
        jnp.dot(a, wn1_ref[...], preferred_element_type=jnp.float32)
        + bn1_ref[...], 0.0)
    o = jnp.dot(z, wn2_ref[...], preferred_element_type=jnp.float32) \
        + bn2_ref[...]
    hout_ref[0] = hb + o


def _node_mlp(others, h, parts, wn1, bn1, wn2, bn2):
    return pl.pallas_call(
        _node_tc,
        grid=(T, N // BN),
        in_specs=[
            pl.BlockSpec((1, BN, H), lambda t, j: (t, j, 0)),
            pl.BlockSpec((1, BN, D), lambda t, j: (t, j, 0)),
            pl.BlockSpec((1, NC, BN, H), lambda t, j: (t, 0, j, 0)),
            pl.BlockSpec((H + D + H, H), lambda t, j: (0, 0)),
            pl.BlockSpec((1, H), lambda t, j: (0, 0)),
            pl.BlockSpec((H, D), lambda t, j: (0, 0)),
            pl.BlockSpec((1, D), lambda t, j: (0, 0)),
        ],
        out_specs=[
            pl.BlockSpec((1, BN, D), lambda t, j: (t, j, 0)),
            pl.BlockSpec((1, BN, H + D + H), lambda t, j: (t, j, 0)),
        ],
        out_shape=[
            jax.ShapeDtypeStruct((T, N, D), jnp.float32),
            jax.ShapeDtypeStruct((T, N, H + D + H), jnp.float32),
        ],
    )(others, h, parts, wn1, bn1, wn2, bn2)


def kernel(h, edge_index, radial, others,
           We1, be1, We2, be2, Wn1, bn1, Wn2, bn2):
    row = edge_index[0]
    col = edge_index[1]
    offs = (jnp.arange(T, dtype=jnp.int32) * N)[:, None]
    ridx = (row[None, :] + offs).reshape(T * E)
    cidx = (col[None, :] + offs).reshape(T * E)
    row3d = row.reshape(NW, NBLK, BB)

    h2d = h.reshape(T * N, D)
    rad2d = radial.reshape(T * E, R)

    p_tab, q_tab = _tables(h2d, We1[:D], We1[D:2 * D])
    g = _gather_add(p_tab, q_tab, ridx, cidx)
    e2 = _edge_mlp(g, rad2d, We1[2 * D:], be1.reshape(1, H), We2,
                   be2.reshape(1, H))
    parts = _scatter(e2, row3d)
    h_out, a_out = _node_mlp(others, h, parts, Wn1, bn1.reshape(1, H), Wn2,
                             bn2.reshape(1, D))
    return h_out, a_out
